# Initial kernel scaffold; baseline (speedup 1.0000x reference)
import jax
import jax.numpy as jnp
from jax.experimental import pallas as pl


def kernel(x, edge_index, W1, b1, W_mu, b_mu, W_ls, b_ls):
    raise NotImplementedError("write your pallas kernel here")


# speedup vs baseline: 24.3401x; 24.3401x over previous
import functools

import jax
import jax.numpy as jnp
from jax import lax
from jax.experimental import pallas as pl
from jax.experimental.pallas import tpu as pltpu
from jax.experimental.pallas import tpu_sc as plsc

N = 10000
E = 320000
C = 128
OUT_CH = 64

NC = 2
NS = 16
NW = NC * NS
EPT = E // NW
CHUNK = 80
NCHUNK = EPT // CHUNK
NPAD = 10240
ROWS_PT = NPAD // NS
ZR = 64
WB = 128
DEG_PT = NPAD // NS

def _zero_vec16():
    return jnp.zeros((16,), jnp.float32)


def _deg_body(dst_hbm, degp_out, dst_v, ones_v, stage_v, deg_sh):
    cid = lax.axis_index("c")
    sid = lax.axis_index("s")
    pltpu.sync_copy(dst_hbm.at[cid, sid], dst_v)
    for i in range(CHUNK // 16):
        ones_v[pl.ds(i * 16, 16)] = jnp.ones((16,), jnp.float32)
    for i in range(DEG_PT // 16):
        stage_v[pl.ds(i * 16, 16)] = _zero_vec16()
    pltpu.sync_copy(stage_v, deg_sh.at[pl.ds(sid * DEG_PT, DEG_PT)])
    plsc.subcore_barrier()

    def body(j, carry):
        pltpu.sync_copy(ones_v, deg_sh.at[dst_v.at[j]], add=True)
        return carry

    lax.fori_loop(0, NCHUNK, body, 0)
    plsc.subcore_barrier()
    pltpu.sync_copy(deg_sh.at[pl.ds(sid * DEG_PT, DEG_PT)], stage_v)
    pltpu.sync_copy(stage_v, degp_out.at[cid, pl.ds(sid * DEG_PT, DEG_PT)])


def _prop_body(table_hbm, eidx_hbm, aggp_out,
               src_v, dst_v, rows_v, acc_sh, sem):
    cid = lax.axis_index("c")
    sid = lax.axis_index("s")
    pltpu.sync_copy(eidx_hbm.at[0, cid, sid], src_v)
    pltpu.sync_copy(eidx_hbm.at[1, cid, sid], dst_v)
    for r in range(CHUNK):
        for k in range(C // 16):
            rows_v[r, pl.ds(k * 16, 16)] = _zero_vec16()
    row0 = sid * ROWS_PT
    for t in range(ROWS_PT // CHUNK):
        pltpu.sync_copy(rows_v, acc_sh.at[pl.ds(row0 + t * CHUNK, CHUNK)])
    plsc.subcore_barrier()

    def body(j, carry):
        pltpu.async_copy(table_hbm.at[src_v.at[j]], rows_v, sem).wait()
        pltpu.sync_copy(rows_v, acc_sh.at[dst_v.at[j]], add=True)
        return carry

    lax.fori_loop(0, NCHUNK, body, 0)
    plsc.subcore_barrier()
    for t in range(ROWS_PT // CHUNK):
        pltpu.sync_copy(acc_sh.at[pl.ds(row0 + t * CHUNK, CHUNK)], rows_v)
        pltpu.sync_copy(rows_v, aggp_out.at[cid, pl.ds(row0 + t * CHUNK, CHUNK)])


@functools.cache
def _sc_kernels():
    mesh = plsc.VectorSubcoreMesh(core_axis_name="c", subcore_axis_name="s")
    deg = pl.kernel(
        _deg_body,
        out_type=jax.ShapeDtypeStruct((NC, NPAD), jnp.float32),
        mesh=mesh,
        scratch_types=[
            pltpu.VMEM((NCHUNK, CHUNK), jnp.int32),
            pltpu.VMEM((CHUNK,), jnp.float32),
            pltpu.VMEM((DEG_PT,), jnp.float32),
            pltpu.VMEM_SHARED((NPAD,), jnp.float32),
        ],
    )
    prop = pl.kernel(
        _prop_body,
        out_type=jax.ShapeDtypeStruct((NC, NPAD, C), jnp.float32),
        mesh=mesh,
        scratch_types=[
            pltpu.VMEM((NCHUNK, CHUNK), jnp.int32),
            pltpu.VMEM((NCHUNK, CHUNK), jnp.int32),
            pltpu.VMEM((CHUNK, C), jnp.float32),
            pltpu.VMEM_SHARED((NPAD, C), jnp.float32),
            pltpu.SemaphoreType.DMA,
        ],
    )
    return deg, prop


def _tc1_body(x_ref, w_ref, degcol_ref, out_ref):
    dinv = lax.rsqrt(degcol_ref[...])
    xw = jnp.dot(x_ref[...], w_ref[...],
                 preferred_element_type=jnp.float32,
                 precision=lax.Precision.HIGHEST)
    out_ref[...] = xw * dinv


def _tc2_body(aggp_ref, scaled1_ref, degcol_ref, b1_ref, wc_ref, out_ref):
    dinv = lax.rsqrt(degcol_ref[...])
    agg = aggp_ref[0, :N] + aggp_ref[1, :N] + scaled1_ref[...]
    h = jnp.maximum(agg * dinv + b1_ref[...], 0.0)
    hw = jnp.dot(h, wc_ref[...],
                 preferred_element_type=jnp.float32,
                 precision=lax.Precision.HIGHEST)
    out_ref[...] = hw * dinv


def _tc3_body(aggp_ref, scaled2_ref, degcol_ref, bc_ref, out_ref):
    dinv = lax.rsqrt(degcol_ref[...])
    out_ref[...] = (aggp_ref[0, :N] + aggp_ref[1, :N] + scaled2_ref[...]) \
        * dinv + bc_ref[...]


_tc1 = pl.pallas_call(_tc1_body, out_shape=jax.ShapeDtypeStruct((N, C), jnp.float32))
_tc2 = pl.pallas_call(_tc2_body, out_shape=jax.ShapeDtypeStruct((N, C), jnp.float32))
_tc3 = pl.pallas_call(_tc3_body, out_shape=jax.ShapeDtypeStruct((N, C), jnp.float32))


def kernel(x, edge_index, W1, b1, W_mu, b_mu, W_ls, b_ls):
    _deg_kernel, _prop_kernel = _sc_kernels()
    e = edge_index.reshape(2, NC, NS, NCHUNK, CHUNK)
    degp = _deg_kernel(e[1])
    degcol = (degp[0, :N] + degp[1, :N] + 1.0).reshape(N, 1)

    scaled1 = _tc1(x, W1, degcol)
    aggp1 = _prop_kernel(scaled1, e)

    Wc = jnp.concatenate([W_mu, W_ls], axis=1)
    bc = jnp.concatenate([b_mu, b_ls]).reshape(1, C)
    scaled2 = _tc2(aggp1, scaled1, degcol, b1.reshape(1, C), Wc)
    aggp2 = _prop_kernel(scaled2, e)

    out = _tc3(aggp2, scaled2, degcol, bc)
    return out[:, :OUT_CH], out[:, OUT_CH:]


# speedup vs baseline: 29.7468x; 1.2221x over previous
import functools

import jax
import jax.numpy as jnp
from jax import lax
from jax.experimental import pallas as pl
from jax.experimental.pallas import tpu as pltpu
from jax.experimental.pallas import tpu_sc as plsc

N = 10000
E = 320000
C = 128
OUT_CH = 64

NC = 2
NS = 16
NW = NC * NS
EPT = E // NW
CHUNK = 80
NCHUNK = EPT // CHUNK
HALF = CHUNK // 2
NPAD = 10240
ROWS_PT = NPAD // NS
ZR = 64
WB = 128
DEG_PT = NPAD // NS

def _zero_vec16():
    return jnp.zeros((16,), jnp.float32)


def _deg_body(dst_hbm, degp_out, dst_v, ones_v, stage_v, deg_sh):
    cid = lax.axis_index("c")
    sid = lax.axis_index("s")
    pltpu.sync_copy(dst_hbm.at[cid, sid], dst_v)
    for i in range(CHUNK // 16):
        ones_v[pl.ds(i * 16, 16)] = jnp.ones((16,), jnp.float32)
    for i in range(DEG_PT // 16):
        stage_v[pl.ds(i * 16, 16)] = _zero_vec16()
    pltpu.sync_copy(stage_v, deg_sh.at[pl.ds(sid * DEG_PT, DEG_PT)])
    plsc.subcore_barrier()

    def body(j, carry):
        pltpu.sync_copy(ones_v, deg_sh.at[dst_v.at[j]], add=True)
        return carry

    lax.fori_loop(0, NCHUNK, body, 0)
    plsc.subcore_barrier()
    pltpu.sync_copy(deg_sh.at[pl.ds(sid * DEG_PT, DEG_PT)], stage_v)
    pltpu.sync_copy(stage_v, degp_out.at[cid, pl.ds(sid * DEG_PT, DEG_PT)])


def _prop_body(table_hbm, eidx_hbm, aggp_out,
               src_v, dst_v, rows_a, rows_b, acc_sh, sem_a, sem_b):
    cid = lax.axis_index("c")
    sid = lax.axis_index("s")
    pltpu.sync_copy(eidx_hbm.at[0, cid, sid], src_v)
    pltpu.sync_copy(eidx_hbm.at[1, cid, sid], dst_v)
    for rv in (rows_a, rows_b):
        for r in range(HALF):
            for k in range(C // 16):
                rv[r, pl.ds(k * 16, 16)] = _zero_vec16()
    row0 = sid * ROWS_PT
    for t in range(ROWS_PT // HALF):
        rv = rows_a if t % 2 == 0 else rows_b
        pltpu.sync_copy(rv, acc_sh.at[pl.ds(row0 + t * HALF, HALF)])
    plsc.subcore_barrier()

    def gather_start(j, off, rv, sem):
        pltpu.async_copy(table_hbm.at[src_v.at[j, pl.ds(off, HALF)]], rv, sem)

    def gather_wait(j, off, rv, sem):
        pltpu.make_async_copy(
            table_hbm.at[src_v.at[j, pl.ds(off, HALF)]], rv, sem).wait()

    gather_start(0, 0, rows_a, sem_a)
    gather_start(0, HALF, rows_b, sem_b)

    def body(j, carry):
        gather_wait(j, 0, rows_a, sem_a)
        pltpu.sync_copy(rows_a, acc_sh.at[dst_v.at[j, pl.ds(0, HALF)]],
                        add=True)
        gather_start(j + 1, 0, rows_a, sem_a)
        gather_wait(j, HALF, rows_b, sem_b)
        pltpu.sync_copy(rows_b, acc_sh.at[dst_v.at[j, pl.ds(HALF, HALF)]],
                        add=True)
        gather_start(j + 1, HALF, rows_b, sem_b)
        return carry

    lax.fori_loop(0, NCHUNK - 1, body, 0)
    last = NCHUNK - 1
    gather_wait(last, 0, rows_a, sem_a)
    pltpu.sync_copy(rows_a, acc_sh.at[dst_v.at[last, pl.ds(0, HALF)]],
                    add=True)
    gather_wait(last, HALF, rows_b, sem_b)
    pltpu.sync_copy(rows_b, acc_sh.at[dst_v.at[last, pl.ds(HALF, HALF)]],
                    add=True)
    plsc.subcore_barrier()
    for t in range(ROWS_PT // HALF):
        rv = rows_a if t % 2 == 0 else rows_b
        pltpu.sync_copy(acc_sh.at[pl.ds(row0 + t * HALF, HALF)], rv)
        pltpu.sync_copy(rv, aggp_out.at[cid, pl.ds(row0 + t * HALF, HALF)])


@functools.cache
def _sc_kernels():
    mesh = plsc.VectorSubcoreMesh(core_axis_name="c", subcore_axis_name="s")
    deg = pl.kernel(
        _deg_body,
        out_type=jax.ShapeDtypeStruct((NC, NPAD), jnp.float32),
        mesh=mesh,
        scratch_types=[
            pltpu.VMEM((NCHUNK, CHUNK), jnp.int32),
            pltpu.VMEM((CHUNK,), jnp.float32),
            pltpu.VMEM((DEG_PT,), jnp.float32),
            pltpu.VMEM_SHARED((NPAD,), jnp.float32),
        ],
    )
    prop = pl.kernel(
        _prop_body,
        out_type=jax.ShapeDtypeStruct((NC, NPAD, C), jnp.float32),
        mesh=mesh,
        scratch_types=[
            pltpu.VMEM((NCHUNK, CHUNK), jnp.int32),
            pltpu.VMEM((NCHUNK, CHUNK), jnp.int32),
            pltpu.VMEM((HALF, C), jnp.float32),
            pltpu.VMEM((HALF, C), jnp.float32),
            pltpu.VMEM_SHARED((NPAD, C), jnp.float32),
            pltpu.SemaphoreType.DMA,
            pltpu.SemaphoreType.DMA,
        ],
    )
    return deg, prop


def _tc1_body(x_ref, w_ref, degcol_ref, out_ref):
    dinv = lax.rsqrt(degcol_ref[...])
    xw = jnp.dot(x_ref[...], w_ref[...],
                 preferred_element_type=jnp.float32,
                 precision=lax.Precision.HIGHEST)
    out_ref[...] = xw * dinv


def _tc2_body(aggp_ref, scaled1_ref, degcol_ref, b1_ref, wc_ref, out_ref):
    dinv = lax.rsqrt(degcol_ref[...])
    agg = aggp_ref[0, :N] + aggp_ref[1, :N] + scaled1_ref[...]
    h = jnp.maximum(agg * dinv + b1_ref[...], 0.0)
    hw = jnp.dot(h, wc_ref[...],
                 preferred_element_type=jnp.float32,
                 precision=lax.Precision.HIGHEST)
    out_ref[...] = hw * dinv


def _tc3_body(aggp_ref, scaled2_ref, degcol_ref, bc_ref, out_ref):
    dinv = lax.rsqrt(degcol_ref[...])
    out_ref[...] = (aggp_ref[0, :N] + aggp_ref[1, :N] + scaled2_ref[...]) \
        * dinv + bc_ref[...]


_tc1 = pl.pallas_call(_tc1_body, out_shape=jax.ShapeDtypeStruct((N, C), jnp.float32))
_tc2 = pl.pallas_call(_tc2_body, out_shape=jax.ShapeDtypeStruct((N, C), jnp.float32))
_tc3 = pl.pallas_call(_tc3_body, out_shape=jax.ShapeDtypeStruct((N, C), jnp.float32))


def kernel(x, edge_index, W1, b1, W_mu, b_mu, W_ls, b_ls):
    _deg_kernel, _prop_kernel = _sc_kernels()
    e = edge_index.reshape(2, NC, NS, NCHUNK, CHUNK)
    degp = _deg_kernel(e[1])
    degcol = (degp[0, :N] + degp[1, :N] + 1.0).reshape(N, 1)

    scaled1 = _tc1(x, W1, degcol)
    aggp1 = _prop_kernel(scaled1, e)

    Wc = jnp.concatenate([W_mu, W_ls], axis=1)
    bc = jnp.concatenate([b_mu, b_ls]).reshape(1, C)
    scaled2 = _tc2(aggp1, scaled1, degcol, b1.reshape(1, C), Wc)
    aggp2 = _prop_kernel(scaled2, e)

    out = _tc3(aggp2, scaled2, degcol, bc)
    return out[:, :OUT_CH], out[:, OUT_CH:]


# speedup vs baseline: 37.0698x; 1.2462x over previous
import functools

import jax
import jax.numpy as jnp
from jax import lax
from jax.experimental import pallas as pl
from jax.experimental.pallas import tpu as pltpu
from jax.experimental.pallas import tpu_sc as plsc

N = 10000
E = 320000
C = 128
OUT_CH = 64

NC = 2
NS = 16
NW = NC * NS
EPT = E // NW
CHUNK = 80
NCHUNK = EPT // CHUNK
HALF = CHUNK // 2
NPAD = 10240
ROWS_PT = NPAD // NS
ZR = 64
WB = 128
DEG_PT = NPAD // NS

def _zero_vec16():
    return jnp.zeros((16,), jnp.float32)


def _deg_body(dst_hbm, degp_out, dst_v, ones_v, stage_v, deg_sh):
    cid = lax.axis_index("c")
    sid = lax.axis_index("s")
    pltpu.sync_copy(dst_hbm.at[cid, sid], dst_v)
    for i in range(CHUNK // 16):
        ones_v[pl.ds(i * 16, 16)] = jnp.ones((16,), jnp.float32)
    for i in range(DEG_PT // 16):
        stage_v[pl.ds(i * 16, 16)] = _zero_vec16()
    pltpu.sync_copy(stage_v, deg_sh.at[pl.ds(sid * DEG_PT, DEG_PT)])
    plsc.subcore_barrier()

    def body(j, carry):
        pltpu.sync_copy(ones_v, deg_sh.at[dst_v.at[j]], add=True)
        return carry

    lax.fori_loop(0, NCHUNK, body, 0)
    plsc.subcore_barrier()
    pltpu.sync_copy(deg_sh.at[pl.ds(sid * DEG_PT, DEG_PT)], stage_v)
    pltpu.sync_copy(stage_v, degp_out.at[cid, pl.ds(sid * DEG_PT, DEG_PT)])


def _prop_body(table_hbm, eidx_hbm, aggp_out,
               src_v, dst_v, rows_0, rows_1, rows_2,
               acc_sh, sem_0, sem_1, sem_2):
    cid = lax.axis_index("c")
    sid = lax.axis_index("s")
    rows = (rows_0, rows_1, rows_2)
    sems = (sem_0, sem_1, sem_2)
    pltpu.sync_copy(eidx_hbm.at[0, cid, sid], src_v)
    pltpu.sync_copy(eidx_hbm.at[1, cid, sid], dst_v)
    for rv in rows:
        for r in range(HALF):
            for k in range(C // 16):
                rv[r, pl.ds(k * 16, 16)] = _zero_vec16()
    row0 = sid * ROWS_PT
    for t in range(ROWS_PT // HALF):
        pltpu.sync_copy(rows[t % 3], acc_sh.at[pl.ds(row0 + t * HALF, HALF)])
    plsc.subcore_barrier()

    def _idx(ref, base_j, k):
        return ref.at[base_j + (k // 2), pl.ds((k % 2) * HALF, HALF)]

    def gather_start(base_j, k, slot):
        pltpu.async_copy(table_hbm.at[_idx(src_v, base_j, k)],
                         rows[slot], sems[slot])

    def gather_wait(base_j, k, slot):
        pltpu.make_async_copy(table_hbm.at[_idx(src_v, base_j, k)],
                              rows[slot], sems[slot]).wait()

    def scatter(base_j, k, slot):
        pltpu.sync_copy(rows[slot], acc_sh.at[_idx(dst_v, base_j, k)],
                        add=True)

    gather_start(0, 0, 0)
    gather_start(0, 1, 1)
    gather_start(0, 2, 2)

    NB = 2 * NCHUNK
    NMAIN = (NB - 4) // 6

    def body(m, carry):
        j0 = 3 * m
        for k in range(6):
            slot = k % 3
            gather_wait(j0, k, slot)
            scatter(j0, k, slot)
            gather_start(j0, k + 3, slot)
        return carry

    lax.fori_loop(0, NMAIN, body, 0)
    je = 3 * NMAIN
    gather_wait(je, 0, 0)
    scatter(je, 0, 0)
    gather_start(je, 3, 0)
    gather_wait(je, 1, 1)
    scatter(je, 1, 1)
    gather_wait(je, 2, 2)
    scatter(je, 2, 2)
    gather_wait(je, 3, 0)
    scatter(je, 3, 0)
    plsc.subcore_barrier()
    for t in range(ROWS_PT // HALF):
        rv = rows[t % 3]
        pltpu.sync_copy(acc_sh.at[pl.ds(row0 + t * HALF, HALF)], rv)
        pltpu.sync_copy(rv, aggp_out.at[cid, pl.ds(row0 + t * HALF, HALF)])


@functools.cache
def _sc_kernels():
    mesh = plsc.VectorSubcoreMesh(core_axis_name="c", subcore_axis_name="s")
    deg = pl.kernel(
        _deg_body,
        out_type=jax.ShapeDtypeStruct((NC, NPAD), jnp.float32),
        mesh=mesh,
        scratch_types=[
            pltpu.VMEM((NCHUNK, CHUNK), jnp.int32),
            pltpu.VMEM((CHUNK,), jnp.float32),
            pltpu.VMEM((DEG_PT,), jnp.float32),
            pltpu.VMEM_SHARED((NPAD,), jnp.float32),
        ],
    )
    prop = pl.kernel(
        _prop_body,
        out_type=jax.ShapeDtypeStruct((NC, NPAD, C), jnp.float32),
        mesh=mesh,
        scratch_types=[
            pltpu.VMEM((NCHUNK, CHUNK), jnp.int32),
            pltpu.VMEM((NCHUNK, CHUNK), jnp.int32),
            pltpu.VMEM((HALF, C), jnp.float32),
            pltpu.VMEM((HALF, C), jnp.float32),
            pltpu.VMEM((HALF, C), jnp.float32),
            pltpu.VMEM_SHARED((NPAD, C), jnp.float32),
            pltpu.SemaphoreType.DMA,
            pltpu.SemaphoreType.DMA,
            pltpu.SemaphoreType.DMA,
        ],
    )
    return deg, prop


def _tc1_body(x_ref, w_ref, degcol_ref, out_ref):
    dinv = lax.rsqrt(degcol_ref[...])
    xw = jnp.dot(x_ref[...], w_ref[...],
                 preferred_element_type=jnp.float32,
                 precision=lax.Precision.HIGHEST)
    out_ref[...] = xw * dinv


def _tc2_body(aggp_ref, scaled1_ref, degcol_ref, b1_ref, wc_ref, out_ref):
    dinv = lax.rsqrt(degcol_ref[...])
    agg = aggp_ref[0, :N] + aggp_ref[1, :N] + scaled1_ref[...]
    h = jnp.maximum(agg * dinv + b1_ref[...], 0.0)
    hw = jnp.dot(h, wc_ref[...],
                 preferred_element_type=jnp.float32,
                 precision=lax.Precision.HIGHEST)
    out_ref[...] = hw * dinv


def _tc3_body(aggp_ref, scaled2_ref, degcol_ref, bc_ref, out_ref):
    dinv = lax.rsqrt(degcol_ref[...])
    out_ref[...] = (aggp_ref[0, :N] + aggp_ref[1, :N] + scaled2_ref[...]) \
        * dinv + bc_ref[...]


_tc1 = pl.pallas_call(_tc1_body, out_shape=jax.ShapeDtypeStruct((N, C), jnp.float32))
_tc2 = pl.pallas_call(_tc2_body, out_shape=jax.ShapeDtypeStruct((N, C), jnp.float32))
_tc3 = pl.pallas_call(_tc3_body, out_shape=jax.ShapeDtypeStruct((N, C), jnp.float32))


def kernel(x, edge_index, W1, b1, W_mu, b_mu, W_ls, b_ls):
    _deg_kernel, _prop_kernel = _sc_kernels()
    e = edge_index.reshape(2, NC, NS, NCHUNK, CHUNK)
    degp = _deg_kernel(e[1])
    degcol = (degp[0, :N] + degp[1, :N] + 1.0).reshape(N, 1)

    scaled1 = _tc1(x, W1, degcol)
    aggp1 = _prop_kernel(scaled1, e)

    Wc = jnp.concatenate([W_mu, W_ls], axis=1)
    bc = jnp.concatenate([b_mu, b_ls]).reshape(1, C)
    scaled2 = _tc2(aggp1, scaled1, degcol, b1.reshape(1, C), Wc)
    aggp2 = _prop_kernel(scaled2, e)

    out = _tc3(aggp2, scaled2, degcol, bc)
    return out[:, :OUT_CH], out[:, OUT_CH:]


# speedup vs baseline: 38.9747x; 1.0514x over previous
import functools

import jax
import jax.numpy as jnp
from jax import lax
from jax.experimental import pallas as pl
from jax.experimental.pallas import tpu as pltpu
from jax.experimental.pallas import tpu_sc as plsc

N = 10000
E = 320000
C = 128
CH = C // 2
OUT_CH = 64

NC = 2
NS = 16
CHUNK = 80
NCK = E // NS // CHUNK
RING = 5
NPAD = 10240
ROWS_PT = NPAD // NS
DEG_CHUNK = 80
DEG_NCK = E // (NC * NS) // DEG_CHUNK
DEG_PT = NPAD // NS


def _zero_vec16():
    return jnp.zeros((16,), jnp.float32)


def _deg_body(dst_hbm, degp_out, dst_v, ones_v, stage_v, deg_sh):
    cid = lax.axis_index("c")
    sid = lax.axis_index("s")
    pltpu.sync_copy(dst_hbm.at[cid, sid], dst_v)
    for i in range(DEG_CHUNK // 16):
        ones_v[pl.ds(i * 16, 16)] = jnp.ones((16,), jnp.float32)
    for i in range(DEG_PT // 16):
        stage_v[pl.ds(i * 16, 16)] = _zero_vec16()
    pltpu.sync_copy(stage_v, deg_sh.at[pl.ds(sid * DEG_PT, DEG_PT)])
    plsc.subcore_barrier()

    def body(j, carry):
        pltpu.sync_copy(ones_v, deg_sh.at[dst_v.at[j]], add=True)
        return carry

    lax.fori_loop(0, DEG_NCK, body, 0)
    plsc.subcore_barrier()
    pltpu.sync_copy(deg_sh.at[pl.ds(sid * DEG_PT, DEG_PT)], stage_v)
    pltpu.sync_copy(stage_v, degp_out.at[cid, pl.ds(sid * DEG_PT, DEG_PT)])


def _prop_body(table_hbm, esrc_hbm, edst_hbm, aggp_out,
               src_v, dst_v, rows_0, rows_1, rows_2, rows_3, rows_4,
               acc_sh, sem_0, sem_1, sem_2, sem_3, sem_4):
    cid = lax.axis_index("c")
    sid = lax.axis_index("s")
    rows = (rows_0, rows_1, rows_2, rows_3, rows_4)
    sems = (sem_0, sem_1, sem_2, sem_3, sem_4)
    pltpu.sync_copy(esrc_hbm.at[cid, sid], src_v)
    pltpu.sync_copy(edst_hbm.at[sid], dst_v)
    for rv in rows:
        for r in range(CHUNK):
            for k in range(CH // 16):
                rv[r, pl.ds(k * 16, 16)] = _zero_vec16()
    row0 = sid * ROWS_PT
    for t in range(ROWS_PT // CHUNK):
        pltpu.sync_copy(rows[t % RING],
                        acc_sh.at[pl.ds(row0 + t * CHUNK, CHUNK)])
    plsc.subcore_barrier()

    def gather_start(j, slot):
        pltpu.async_copy(table_hbm.at[src_v.at[j]], rows[slot], sems[slot])

    def gather_wait(j, slot):
        pltpu.make_async_copy(table_hbm.at[src_v.at[j]],
                              rows[slot], sems[slot]).wait()

    def scatter(j, slot):
        pltpu.sync_copy(rows[slot], acc_sh.at[dst_v.at[j]], add=True)

    for slot in range(RING):
        gather_start(slot, slot)

    def body(m, carry):
        j0 = RING * m
        for k in range(RING):
            gather_wait(j0 + k, k)
            scatter(j0 + k, k)
            gather_start(j0 + k + RING, k)
        return carry

    NMAIN = NCK // RING - 1
    lax.fori_loop(0, NMAIN, body, 0)
    je = RING * NMAIN
    for k in range(RING):
        gather_wait(je + k, k)
        scatter(je + k, k)
    plsc.subcore_barrier()
    for t in range(ROWS_PT // CHUNK):
        rv = rows[t % RING]
        pltpu.sync_copy(acc_sh.at[pl.ds(row0 + t * CHUNK, CHUNK)], rv)
        pltpu.sync_copy(rv, aggp_out.at[cid, pl.ds(row0 + t * CHUNK, CHUNK)])


@functools.cache
def _sc_kernels():
    mesh = plsc.VectorSubcoreMesh(core_axis_name="c", subcore_axis_name="s")
    deg = pl.kernel(
        _deg_body,
        out_type=jax.ShapeDtypeStruct((NC, NPAD), jnp.float32),
        mesh=mesh,
        scratch_types=[
            pltpu.VMEM((DEG_NCK, DEG_CHUNK), jnp.int32),
            pltpu.VMEM((DEG_CHUNK,), jnp.float32),
            pltpu.VMEM((DEG_PT,), jnp.float32),
            pltpu.VMEM_SHARED((NPAD,), jnp.float32),
        ],
    )
    prop = pl.kernel(
        _prop_body,
        out_type=jax.ShapeDtypeStruct((NC, NPAD, CH), jnp.float32),
        mesh=mesh,
        compiler_params=pltpu.CompilerParams(use_tc_tiling_on_sc=False),
        scratch_types=[
            pltpu.VMEM((NCK, CHUNK), jnp.int32),
            pltpu.VMEM((NCK, CHUNK), jnp.int32),
            pltpu.VMEM((CHUNK, CH), jnp.float32),
            pltpu.VMEM((CHUNK, CH), jnp.float32),
            pltpu.VMEM((CHUNK, CH), jnp.float32),
            pltpu.VMEM((CHUNK, CH), jnp.float32),
            pltpu.VMEM((CHUNK, CH), jnp.float32),
            pltpu.VMEM_SHARED((NPAD, CH), jnp.float32),
            pltpu.SemaphoreType.DMA,
            pltpu.SemaphoreType.DMA,
            pltpu.SemaphoreType.DMA,
            pltpu.SemaphoreType.DMA,
            pltpu.SemaphoreType.DMA,
        ],
    )
    return deg, prop


def _tc1_body(x_ref, w_ref, degcol_ref, out_ref):
    dinv = lax.rsqrt(degcol_ref[...])
    xw = jnp.dot(x_ref[...], w_ref[...],
                 preferred_element_type=jnp.float32,
                 precision=lax.Precision.HIGHEST)
    out_ref[...] = xw * dinv


def _tc2_body(agg_ref, scaled1_ref, degcol_ref, b1_ref, wc_ref, out_ref):
    dinv = lax.rsqrt(degcol_ref[...])
    h = jnp.maximum((agg_ref[...] + scaled1_ref[...]) * dinv + b1_ref[...],
                    0.0)
    hw = jnp.dot(h, wc_ref[...],
                 preferred_element_type=jnp.float32,
                 precision=lax.Precision.HIGHEST)
    out_ref[...] = hw * dinv


def _tc3_body(agg_ref, scaled2_ref, degcol_ref, bc_ref, out_ref):
    dinv = lax.rsqrt(degcol_ref[...])
    out_ref[...] = (agg_ref[...] + scaled2_ref[...]) * dinv + bc_ref[...]


_tc1 = pl.pallas_call(
    _tc1_body, out_shape=jax.ShapeDtypeStruct((N, C), jnp.float32))
_tc2 = pl.pallas_call(
    _tc2_body, out_shape=jax.ShapeDtypeStruct((N, C), jnp.float32))
_tc3 = pl.pallas_call(
    _tc3_body, out_shape=jax.ShapeDtypeStruct((N, C), jnp.float32))


def _cat_halves(aggp):
    return jnp.concatenate([aggp[0, :N], aggp[1, :N]], axis=1)


def kernel(x, edge_index, W1, b1, W_mu, b_mu, W_ls, b_ls):
    _deg_kernel, _prop_kernel = _sc_kernels()
    src = edge_index[0].reshape(NS, NCK, CHUNK)
    esrc = jnp.stack([2 * src, 2 * src + 1])
    edst = edge_index[1].reshape(NS, NCK, CHUNK)
    e_deg = edge_index[1].reshape(NC, NS, DEG_NCK, DEG_CHUNK)

    degp = _deg_kernel(e_deg)
    degcol = (degp[0, :N] + degp[1, :N] + 1.0).reshape(N, 1)

    scaled1 = _tc1(x, W1, degcol)
    aggp1 = _prop_kernel(scaled1.reshape(2 * N, CH), esrc, edst)

    Wc = jnp.concatenate([W_mu, W_ls], axis=1)
    bc = jnp.concatenate([b_mu, b_ls]).reshape(1, C)
    t2 = _tc2(_cat_halves(aggp1), scaled1, degcol, b1.reshape(1, C), Wc)
    aggp2 = _prop_kernel(t2.reshape(2 * N, CH), esrc, edst)

    out = _tc3(_cat_halves(aggp2), t2, degcol, bc)
    return out[:, :OUT_CH], out[:, OUT_CH:]


# speedup vs baseline: 39.3693x; 1.0101x over previous
import functools

import jax
import jax.numpy as jnp
from jax import lax
from jax.experimental import pallas as pl
from jax.experimental.pallas import tpu as pltpu
from jax.experimental.pallas import tpu_sc as plsc

N = 10000
E = 320000
C = 128
CH = C // 2
OUT_CH = 64

NC = 2
NS = 16
CHUNK = 80
NCK = E // NS // CHUNK
RING = 5
NPAD = 10240
ROWS_PT = NPAD // NS
DEG_CHUNK = 80
DEG_NCK = E // (NC * NS) // DEG_CHUNK
DEG_PT = NPAD // NS


def _zero_vec16():
    return jnp.zeros((16,), jnp.float32)


def _deg_body(dst_hbm, degp_out, dst_v, ones_v, stage_v, deg_sh):
    cid = lax.axis_index("c")
    sid = lax.axis_index("s")
    pltpu.sync_copy(dst_hbm.at[cid, sid], dst_v)
    for i in range(DEG_CHUNK // 16):
        ones_v[pl.ds(i * 16, 16)] = jnp.ones((16,), jnp.float32)
    for i in range(DEG_PT // 16):
        stage_v[pl.ds(i * 16, 16)] = _zero_vec16()
    pltpu.sync_copy(stage_v, deg_sh.at[pl.ds(sid * DEG_PT, DEG_PT)])
    plsc.subcore_barrier()

    def body(j, carry):
        pltpu.sync_copy(ones_v, deg_sh.at[dst_v.at[j]], add=True)
        return carry

    lax.fori_loop(0, DEG_NCK, body, 0)
    plsc.subcore_barrier()
    pltpu.sync_copy(deg_sh.at[pl.ds(sid * DEG_PT, DEG_PT)], stage_v)
    pltpu.sync_copy(stage_v, degp_out.at[cid, pl.ds(sid * DEG_PT, DEG_PT)])


def _prop_body(table_hbm, esrc_hbm, edst_hbm, aggp_out,
               src_v, dst_v, rows_0, rows_1, rows_2, rows_3, rows_4,
               acc_sh, sem_0, sem_1, sem_2, sem_3, sem_4):
    cid = lax.axis_index("c")
    sid = lax.axis_index("s")
    rows = (rows_0, rows_1, rows_2, rows_3, rows_4)
    sems = (sem_0, sem_1, sem_2, sem_3, sem_4)
    pltpu.sync_copy(esrc_hbm.at[cid, sid], src_v)
    pltpu.sync_copy(edst_hbm.at[sid], dst_v)
    for rv in rows:
        for r in range(CHUNK):
            for k in range(CH // 16):
                rv[r, pl.ds(k * 16, 16)] = _zero_vec16()
    row0 = sid * ROWS_PT
    NWB = ROWS_PT // CHUNK

    def _acc_slice(t):
        return acc_sh.at[pl.ds(row0 + t * CHUNK, CHUNK)]

    for t in range(NWB):
        pltpu.async_copy(rows[t % RING], _acc_slice(t), sems[0])
    for t in range(NWB):
        pltpu.make_async_copy(rows[t % RING], _acc_slice(t), sems[0]).wait()
    plsc.subcore_barrier()

    def gather_start(j, slot):
        pltpu.async_copy(table_hbm.at[src_v.at[j]], rows[slot], sems[slot])

    def gather_wait(j, slot):
        pltpu.make_async_copy(table_hbm.at[src_v.at[j]],
                              rows[slot], sems[slot]).wait()

    def scatter(j, slot):
        pltpu.sync_copy(rows[slot], acc_sh.at[dst_v.at[j]], add=True)

    for slot in range(RING):
        gather_start(slot, slot)

    def body(m, carry):
        j0 = RING * m
        for k in range(RING):
            gather_wait(j0 + k, k)
            scatter(j0 + k, k)
            gather_start(j0 + k + RING, k)
        return carry

    NMAIN = NCK // RING - 1
    lax.fori_loop(0, NMAIN, body, 0)
    je = RING * NMAIN
    for k in range(RING):
        gather_wait(je + k, k)
        scatter(je + k, k)
    plsc.subcore_barrier()

    def _out_slice(t):
        return aggp_out.at[cid, pl.ds(row0 + t * CHUNK, CHUNK)]

    for t in range(NWB):
        if t >= RING:
            s = t - RING
            pltpu.make_async_copy(rows[s % RING], _out_slice(s),
                                  sems[s % RING]).wait()
        pltpu.sync_copy(acc_sh.at[pl.ds(row0 + t * CHUNK, CHUNK)],
                        rows[t % RING])
        pltpu.async_copy(rows[t % RING], _out_slice(t), sems[t % RING])
    for t in range(max(0, NWB - RING), NWB):
        pltpu.make_async_copy(rows[t % RING], _out_slice(t),
                              sems[t % RING]).wait()


@functools.cache
def _sc_kernels():
    mesh = plsc.VectorSubcoreMesh(core_axis_name="c", subcore_axis_name="s")
    deg = pl.kernel(
        _deg_body,
        out_type=jax.ShapeDtypeStruct((NC, NPAD), jnp.float32),
        mesh=mesh,
        scratch_types=[
            pltpu.VMEM((DEG_NCK, DEG_CHUNK), jnp.int32),
            pltpu.VMEM((DEG_CHUNK,), jnp.float32),
            pltpu.VMEM((DEG_PT,), jnp.float32),
            pltpu.VMEM_SHARED((NPAD,), jnp.float32),
        ],
    )
    prop = pl.kernel(
        _prop_body,
        out_type=jax.ShapeDtypeStruct((NC, NPAD, CH), jnp.float32),
        mesh=mesh,
        compiler_params=pltpu.CompilerParams(use_tc_tiling_on_sc=False),
        scratch_types=[
            pltpu.VMEM((NCK, CHUNK), jnp.int32),
            pltpu.VMEM((NCK, CHUNK), jnp.int32),
            pltpu.VMEM((CHUNK, CH), jnp.float32),
            pltpu.VMEM((CHUNK, CH), jnp.float32),
            pltpu.VMEM((CHUNK, CH), jnp.float32),
            pltpu.VMEM((CHUNK, CH), jnp.float32),
            pltpu.VMEM((CHUNK, CH), jnp.float32),
            pltpu.VMEM_SHARED((NPAD, CH), jnp.float32),
            pltpu.SemaphoreType.DMA,
            pltpu.SemaphoreType.DMA,
            pltpu.SemaphoreType.DMA,
            pltpu.SemaphoreType.DMA,
            pltpu.SemaphoreType.DMA,
        ],
    )
    return deg, prop


def _tc1_body(x_ref, w_ref, degcol_ref, out_ref):
    dinv = lax.rsqrt(degcol_ref[...])
    xw = jnp.dot(x_ref[...], w_ref[...],
                 preferred_element_type=jnp.float32,
                 precision=lax.Precision.HIGHEST)
    out_ref[...] = xw * dinv


def _tc2_body(agg_ref, scaled1_ref, degcol_ref, b1_ref, wc_ref, out_ref):
    dinv = lax.rsqrt(degcol_ref[...])
    h = jnp.maximum((agg_ref[...] + scaled1_ref[...]) * dinv + b1_ref[...],
                    0.0)
    hw = jnp.dot(h, wc_ref[...],
                 preferred_element_type=jnp.float32,
                 precision=lax.Precision.HIGHEST)
    out_ref[...] = hw * dinv


def _tc3_body(agg_ref, scaled2_ref, degcol_ref, bc_ref, out_ref):
    dinv = lax.rsqrt(degcol_ref[...])
    out_ref[...] = (agg_ref[...] + scaled2_ref[...]) * dinv + bc_ref[...]


_tc1 = pl.pallas_call(
    _tc1_body, out_shape=jax.ShapeDtypeStruct((N, C), jnp.float32))
_tc2 = pl.pallas_call(
    _tc2_body, out_shape=jax.ShapeDtypeStruct((N, C), jnp.float32))
_tc3 = pl.pallas_call(
    _tc3_body, out_shape=jax.ShapeDtypeStruct((N, C), jnp.float32))


def _cat_halves(aggp):
    return jnp.concatenate([aggp[0, :N], aggp[1, :N]], axis=1)


def kernel(x, edge_index, W1, b1, W_mu, b_mu, W_ls, b_ls):
    _deg_kernel, _prop_kernel = _sc_kernels()
    src = edge_index[0].reshape(NS, NCK, CHUNK)
    esrc = jnp.stack([2 * src, 2 * src + 1])
    edst = edge_index[1].reshape(NS, NCK, CHUNK)
    e_deg = edge_index[1].reshape(NC, NS, DEG_NCK, DEG_CHUNK)

    degp = _deg_kernel(e_deg)
    degcol = (degp[0, :N] + degp[1, :N] + 1.0).reshape(N, 1)

    scaled1 = _tc1(x, W1, degcol)
    aggp1 = _prop_kernel(scaled1.reshape(2 * N, CH), esrc, edst)

    Wc = jnp.concatenate([W_mu, W_ls], axis=1)
    bc = jnp.concatenate([b_mu, b_ls]).reshape(1, C)
    t2 = _tc2(_cat_halves(aggp1), scaled1, degcol, b1.reshape(1, C), Wc)
    aggp2 = _prop_kernel(t2.reshape(2 * N, CH), esrc, edst)

    out = _tc3(_cat_halves(aggp2), t2, degcol, bc)
    return out[:, :OUT_CH], out[:, OUT_CH:]
